# Initial kernel scaffold; baseline (speedup 1.0000x reference)
import jax
import jax.numpy as jnp
from jax.experimental import pallas as pl


def kernel(x, gate_w, gate_b, expert_params):
    raise NotImplementedError("write your pallas kernel here")


# speedup vs baseline: 2.9160x; 2.9160x over previous
import functools

import jax
import jax.numpy as jnp
from jax.experimental import pallas as pl
import jax.experimental.pallas.tpu as pltpu

_EPSBN = 1e-5
_NE = 8
_B = 64
_GS = 8
_NG = 15
_NS = _NG * _GS


def _bn_sb(bn):
    sc = bn[0] * jax.lax.rsqrt(bn[3] + _EPSBN)
    bi = bn[1] - bn[2] * sc
    return sc, bi


def _conv_chw(X, w, bn, G, H, W, relu=True, res=None):
    C = X.shape[0]
    L = G * H * W
    p = jax.lax.broadcasted_iota(jnp.int32, (1, L), 1)
    hp = (p // W) % H
    wp = p % W
    zp = jnp.zeros((C, W + 1), jnp.float32)
    Xp = jnp.concatenate([zp, X, zp], axis=1)
    cols = []
    for dh in (-1, 0, 1):
        for dw in (-1, 0, 1):
            off = (W + 1) + dh * W + dw
            s = jax.lax.slice(Xp, (0, off), (C, off + L))
            m = ((hp + dh >= 0) & (hp + dh < H)
                 & (wp + dw >= 0) & (wp + dw < W))
            cols.append(jnp.where(m, s, 0.0))
    im = jnp.concatenate(cols, axis=0)
    y = jax.lax.dot_general(w, im, (((1,), (0,)), ((), ())),
                            preferred_element_type=jnp.float32)
    sc, bi = _bn_sb(bn)
    y = y * sc[:, None] + bi[:, None]
    if res is not None:
        y = y + res
    return jnp.maximum(y, 0.0) if relu else y


def _sub2_nhwc(X):
    G, H, W, C = X.shape
    return X.reshape(G, H // 2, 2, W // 2, 2, C)[:, :, 0, :, 0, :]


def _conv_nhwc(X, w, bn, stride=1, relu=True, res=None):
    G, H, W, C = X.shape
    zH = jnp.zeros((G, 1, W, C), jnp.float32)
    Xp = jnp.concatenate([zH, X, zH], axis=1)
    zW = jnp.zeros((G, H + 2, 1, C), jnp.float32)
    Xp = jnp.concatenate([zW, Xp, zW], axis=2)
    cols = [Xp[:, dh:dh + H, dw:dw + W, :]
            for dh in range(3) for dw in range(3)]
    im = jnp.concatenate(cols, axis=-1)
    if stride == 2:
        im = _sub2_nhwc(im)
        H, W = H // 2, W // 2
    Cout = w.shape[1]
    y = jax.lax.dot_general(im.reshape(G * H * W, 9 * C), w,
                            (((1,), (0,)), ((), ())),
                            preferred_element_type=jnp.float32)
    sc, bi = _bn_sb(bn)
    y = y * sc[None, :] + bi[None, :]
    y = y.reshape(G, H, W, Cout)
    if res is not None:
        y = y + res
    return jnp.maximum(y, 0.0) if relu else y


def _down_nhwc(X, w, bn):
    G, H, W, C = X.shape
    Xs = _sub2_nhwc(X)
    y = jax.lax.dot_general(Xs.reshape(G * H * W // 4, C), w,
                            (((1,), (0,)), ((), ())),
                            preferred_element_type=jnp.float32)
    sc, bi = _bn_sb(bn)
    y = y * sc[None, :] + bi[None, :]
    return y.reshape(G, H // 2, W // 2, w.shape[1])


def _block_nhwc(X, wc1, bn1, wc2, bn2, wd=None, bnd=None, stride=1):
    out = _conv_nhwc(X, wc1, bn1, stride=stride, relu=True)
    if wd is not None:
        sc = _down_nhwc(X, wd, bnd)
    else:
        sc = X
    return _conv_nhwc(out, wc2, bn2, stride=1, relu=True, res=sc)


def _router_body(xf_ref, gw_ref, gb_ref, e_ref, slots_ref, pos_ref):
    logits = jax.lax.dot_general(
        xf_ref[...], gw_ref[...], (((1,), (1,)), ((), ())),
        preferred_element_type=jnp.float32,
        precision=jax.lax.Precision.HIGHEST) + gb_ref[...]
    i8 = jax.lax.broadcasted_iota(jnp.int32, (_B, _NE), 1)
    mx = jnp.max(logits, axis=1, keepdims=True)
    cand = jnp.where(logits >= mx, i8, _NE)
    route = jnp.min(cand, axis=1, keepdims=True)
    onehot = (i8 == route).astype(jnp.float32)
    r64 = jax.lax.broadcasted_iota(jnp.int32, (_B, _B), 0)
    c64 = jax.lax.broadcasted_iota(jnp.int32, (_B, _B), 1)
    M = (c64 <= r64).astype(jnp.float32)
    csum = jax.lax.dot_general(M, onehot, (((1,), (0,)), ((), ())),
                               preferred_element_type=jnp.float32)
    rank = jnp.sum(csum * onehot, axis=1, keepdims=True) - 1.0
    counts = jnp.sum(onehot, axis=0, keepdims=True)
    gcnt = jnp.floor((counts + 7.0) * 0.125)
    r8 = jax.lax.broadcasted_iota(jnp.int32, (_NE, _NE), 0)
    c8 = jax.lax.broadcasted_iota(jnp.int32, (_NE, _NE), 1)
    U = (r8 < c8).astype(jnp.float32)
    gbase = 8.0 * jax.lax.dot_general(gcnt, U, (((1,), (0,)), ((), ())),
                                      preferred_element_type=jnp.float32)
    pos = jax.lax.dot_general(
        onehot, gbase.reshape(_NE, 1), (((1,), (0,)), ((), ())),
        preferred_element_type=jnp.float32) + rank
    g16 = jax.lax.broadcasted_iota(jnp.int32, (16, _NE), 0).astype(
        jnp.float32) * 8.0
    e16 = jax.lax.broadcasted_iota(jnp.int32, (16, _NE), 1).astype(
        jnp.float32)
    match = ((g16 >= gbase) & (g16 < gbase + 8.0 * gcnt)).astype(jnp.float32)
    ematch = jnp.sum(match * e16, axis=1, keepdims=True)
    anym = jnp.sum(match, axis=1, keepdims=True)
    eids = jax.lax.broadcasted_iota(jnp.int32, (1, _NE), 1).astype(
        jnp.float32)
    last_used = jnp.max(jnp.where(counts > 0, eids, -1.0), axis=1,
                        keepdims=True)
    etab = jnp.where(anym > 0, ematch, last_used)
    s120 = jax.lax.broadcasted_iota(jnp.int32, (_NS, _B), 0).astype(
        jnp.float32)
    i120 = jax.lax.broadcasted_iota(jnp.int32, (_NS, _B), 1).astype(
        jnp.float32)
    msl = (s120 == pos.reshape(1, _B)).astype(jnp.float32)
    img = jnp.sum(msl * i120, axis=1, keepdims=True)
    e_ref[...] = etab.astype(jnp.int32)
    slots_ref[...] = img.astype(jnp.int32)
    pos_ref[...] = pos.astype(jnp.int32)


def _stageA_body_real(e_ref, slots_ref, *refs):
    xs = refs[0:_GS]
    (wc1, bnc1, w0a1, bn0a1, w0a2, bn0a2,
     w0b1, bn0b1, w0b2, bn0b2, o_ref) = refs[_GS:]
    X = jnp.concatenate([r[0] for r in xs], axis=1)
    G, H, W = _GS, 32, 32
    h0 = _conv_chw(X, wc1[0], bnc1[0], G, H, W, relu=True)
    t = _conv_chw(h0, w0a1[0], bn0a1[0], G, H, W, relu=True)
    h1 = _conv_chw(t, w0a2[0], bn0a2[0], G, H, W, relu=True, res=h0)
    t = _conv_chw(h1, w0b1[0], bn0b1[0], G, H, W, relu=True)
    h2 = _conv_chw(t, w0b2[0], bn0b2[0], G, H, W, relu=True, res=h1)
    o_ref[0] = h2


def _stageB_body(e_ref, a_ref, wc1, bn1, wc2, bn2, wd, bnd,
                 wc3, bn3, wc4, bn4, o_ref):
    X = a_ref[...]
    h = _block_nhwc(X, wc1[0], bn1[0], wc2[0], bn2[0],
                    wd[0], bnd[0], stride=2)
    h = _block_nhwc(h, wc3[0], bn3[0], wc4[0], bn4[0])
    o_ref[...] = h


def _stageD1_body(e_ref, a_ref, wc1, bn1, wc2, bn2, wd, bnd, o_ref):
    X = a_ref[...]
    o_ref[...] = _block_nhwc(X, wc1[0], bn1[0], wc2[0], bn2[0],
                             wd[0], bnd[0], stride=2)


def _stageD2_body(e_ref, a_ref, wc3, bn3, wc4, bn4, fw, fb, o_ref):
    X = a_ref[...]
    h = _block_nhwc(X, wc3[0], bn3[0], wc4[0], bn4[0])
    h = jnp.sum(jnp.sum(h, axis=1), axis=1) * (1.0 / 16.0)
    y = jax.lax.dot_general(h, fw[0], (((1,), (0,)), ((), ())),
                            preferred_element_type=jnp.float32)
    o_ref[...] = y + fb[0]


def _gather_body(so_ref, pos_ref, o_ref):
    pos = pos_ref[...]
    s = jax.lax.broadcasted_iota(jnp.int32, (_B, _NS), 1)
    onehot = (s == pos).astype(jnp.float32)
    o_ref[...] = jax.lax.dot_general(
        onehot, so_ref[...], (((1,), (0,)), ((), ())),
        preferred_element_type=jnp.float32)


def _wchw(w):
    E, O, I = w.shape[0], w.shape[1], w.shape[2]
    return w.transpose(0, 1, 3, 4, 2).reshape(E, O, 9 * I)


def _wnhwc(w):
    E, O, I = w.shape[0], w.shape[1], w.shape[2]
    return w.transpose(0, 3, 4, 2, 1).reshape(E, 9 * I, O)


def _wdown(w):
    return w[:, :, :, 0, 0].transpose(0, 2, 1)


def _bnp(bn):
    return jnp.stack([bn['gamma'], bn['beta'], bn['mean'], bn['var']],
                     axis=1)


def _wspec(arr):
    shp = (1,) + arr.shape[1:]
    nd = len(shp)
    def imap(g, E, *rest):
        return (E[g],) + (0,) * (nd - 1)
    return pl.BlockSpec(shp, imap)


def _wspec1(arr):
    shp = (1,) + arr.shape[1:]
    nd = len(shp)
    def imap(g, E):
        return (E[g],) + (0,) * (nd - 1)
    return pl.BlockSpec(shp, imap)


def kernel(x, gate_w, gate_b, expert_params):
    p = expert_params
    xf = x.reshape(_B, 3 * 32 * 32)

    etab, slots, pos = pl.pallas_call(
        _router_body,
        out_shape=(jax.ShapeDtypeStruct((16, 1), jnp.int32),
                   jax.ShapeDtypeStruct((_NS, 1), jnp.int32),
                   jax.ShapeDtypeStruct((_B, 1), jnp.int32)),
    )(xf, gate_w, gate_b.reshape(1, _NE))
    etab1 = etab.reshape(16)
    slots1 = slots.reshape(_NS)

    xc = x.reshape(_B, 3, 1024)
    l0 = p['layer0']
    a_ins = [xc] * _GS + [
        _wchw(p['conv1']), _bnp(p['bn1']),
        _wchw(l0[0]['conv1']), _bnp(l0[0]['bn1']),
        _wchw(l0[0]['conv2']), _bnp(l0[0]['bn2']),
        _wchw(l0[1]['conv1']), _bnp(l0[1]['bn1']),
        _wchw(l0[1]['conv2']), _bnp(l0[1]['bn2']),
    ]
    x_specs = []
    for j in range(_GS):
        def imap(g, E, S, j=j):
            return (S[g * _GS + j], 0, 0)
        x_specs.append(pl.BlockSpec((1, 3, 1024), imap))
    a_specs = x_specs + [_wspec(a) for a in a_ins[_GS:]]
    actA = pl.pallas_call(
        _stageA_body_real,
        grid_spec=pltpu.PrefetchScalarGridSpec(
            num_scalar_prefetch=2,
            grid=(_NG,),
            in_specs=a_specs,
            out_specs=pl.BlockSpec((1, 64, _GS * 1024),
                                   lambda g, E, S: (g, 0, 0)),
        ),
        out_shape=jax.ShapeDtypeStruct((_NG, 64, _GS * 1024), jnp.float32),
    )(etab1, slots1, *a_ins)

    actA = actA.reshape(_NG, 64, _GS, 32, 32).transpose(0, 2, 3, 4, 1)
    actA = actA.reshape(_NS, 32, 32, 64)

    def stage(body, act, wlist, out_shape, extra=()):
        ins = list(wlist) + list(extra)
        in_specs = [pl.BlockSpec((_GS,) + act.shape[1:],
                                 lambda g, E: (g,) + (0,) * (len(act.shape) - 1))]
        in_specs += [_wspec1(a) for a in ins]
        return pl.pallas_call(
            body,
            grid_spec=pltpu.PrefetchScalarGridSpec(
                num_scalar_prefetch=1,
                grid=(_NG,),
                in_specs=in_specs,
                out_specs=pl.BlockSpec(
                    (_GS,) + out_shape[1:],
                    lambda g, E: (g,) + (0,) * (len(out_shape) - 1)),
            ),
            out_shape=jax.ShapeDtypeStruct((_NS,) + out_shape[1:],
                                           jnp.float32),
        )(etab1, act, *ins)

    def layer_w(l):
        b0, b1 = l[0], l[1]
        return [_wnhwc(b0['conv1']), _bnp(b0['bn1']),
                _wnhwc(b0['conv2']), _bnp(b0['bn2']),
                _wdown(b0['down_conv']), _bnp(b0['down_bn']),
                _wnhwc(b1['conv1']), _bnp(b1['bn1']),
                _wnhwc(b1['conv2']), _bnp(b1['bn2'])]

    actB = stage(_stageB_body, actA, layer_w(p['layer1']),
                 (_NS, 16, 16, 128))
    actC = stage(_stageB_body, actB, layer_w(p['layer2']),
                 (_NS, 8, 8, 256))
    fc_w = p['fc_w'].transpose(0, 2, 1)
    fc_b = p['fc_b'].reshape(_NE, 1, 10)
    l3 = layer_w(p['layer3'])
    actD1 = stage(_stageD1_body, actC, l3[:6], (_NS, 4, 4, 512))
    actD = stage(_stageD2_body, actD1, l3[6:],
                 (_NS, 10), extra=(fc_w, fc_b))

    out = pl.pallas_call(
        _gather_body,
        out_shape=jax.ShapeDtypeStruct((_B, 10), jnp.float32),
    )(actD, pos)
    return out
